# baseline (speedup 1.0000x reference)
import jax
import jax.numpy as jnp
from jax.experimental import pallas as pl
from jax.experimental.pallas import tpu as pltpu

B, H, N, D = 8, 12, 577, 768
K = 288
NP = N - 1
HH = 6

_T_DN = (((0,), (0,)), ((), ()))


def _merge_kernel(a_ref, tokens_ref, merged_ref, idx_ref, m_ref):
    j = pl.program_id(1)
    w = a_ref[0]
    half = (((w[0] + (w[1] + w[2])) + w[3]) + w[4]) + w[5]

    @pl.when(j == 0)
    def _first():
        m_ref[...] = half

    @pl.when(j == 1)
    def _second():
        m_ref[...] = (m_ref[...] + half) * (jnp.float32(1) / jnp.float32(H))

        mass = m_ref[0:1, :]
        for q in range(1, N):
            mass = mass + m_ref[q:q + 1, :]

        pw = mass[:, 1:N]
        ones = jnp.ones((1, NP), jnp.float32)
        vcol = jax.lax.dot_general(
            pw, ones, _T_DN,
            precision=jax.lax.Precision.HIGHEST,
            preferred_element_type=jnp.float32,
        )
        vrow = jnp.broadcast_to(pw, (NP, NP))
        jj = jax.lax.broadcasted_iota(jnp.int32, (NP, NP), 1)
        ii = jax.lax.broadcasted_iota(jnp.int32, (NP, NP), 0)
        beats = (vrow > vcol) | ((vrow == vcol) & (jj < ii))
        rank = jnp.sum(beats.astype(jnp.int32), axis=1, keepdims=True)

        rr = jax.lax.broadcasted_iota(jnp.int32, (NP, K), 1)
        sel_mask = rank == rr
        iidx = jax.lax.broadcasted_iota(jnp.int32, (NP, K), 0)
        idx_ref[0, :] = jnp.sum(jnp.where(sel_mask, iidx, 0), axis=0)[None, :]

        mask16 = sel_mask.astype(jnp.bfloat16)
        patches = tokens_ref[0, 1:N, :]
        hi = patches.astype(jnp.bfloat16)
        rest = (patches - hi.astype(jnp.float32)).astype(jnp.bfloat16)
        sel = jax.lax.dot_general(
            mask16, hi, _T_DN, preferred_element_type=jnp.float32,
        ) + jax.lax.dot_general(
            mask16, rest, _T_DN, preferred_element_type=jnp.float32,
        )
        merged_ref[0, 0] = tokens_ref[0, 0]
        merged_ref[0, 1:K + 1, :] = sel


@jax.jit
def kernel(tokens, attention_maps):
    merged, idx = pl.pallas_call(
        _merge_kernel,
        grid=(B, 2),
        in_specs=[
            pl.BlockSpec((1, HH, N, N), lambda b, j: (b, j, 0, 0)),
            pl.BlockSpec((1, N, D), lambda b, j: (b, 0, 0)),
        ],
        out_specs=[
            pl.BlockSpec((1, K + 1, D), lambda b, j: (b, 0, 0)),
            pl.BlockSpec((1, 1, K), lambda b, j: (b, 0, 0)),
        ],
        out_shape=[
            jax.ShapeDtypeStruct((B, K + 1, D), jnp.float32),
            jax.ShapeDtypeStruct((B, 1, K), jnp.int32),
        ],
        scratch_shapes=[
            pltpu.VMEM((N, N), jnp.float32),
        ],
        compiler_params=pltpu.CompilerParams(
            dimension_semantics=("arbitrary", "arbitrary"),
        ),
    )(attention_maps, tokens)
    return merged, idx.reshape(B, K)


# speedup vs baseline: 1.0924x; 1.0924x over previous
import jax
import jax.numpy as jnp
from jax.experimental import pallas as pl
from jax.experimental.pallas import tpu as pltpu

B, H, N, D = 8, 12, 577, 768
K = 288
NP = N - 1

_T_DN = (((0,), (0,)), ((), ()))


def _merge_kernel(a_ref, tokens_ref, merged_ref, idx_ref, m_ref):
    a = a_ref[0]
    half1 = (((a[0] + (a[1] + a[2])) + a[3]) + a[4]) + a[5]
    half2 = (((a[6] + (a[7] + a[8])) + a[9]) + a[10]) + a[11]
    m_ref[...] = (half1 + half2) * (jnp.float32(1) / jnp.float32(H))

    mass = m_ref[0:1, :]
    for q in range(1, N):
        mass = mass + m_ref[q:q + 1, :]

    pw = mass[:, 1:N]
    ones = jnp.ones((1, NP), jnp.float32)
    vcol = jax.lax.dot_general(
        pw, ones, _T_DN,
        precision=jax.lax.Precision.HIGHEST,
        preferred_element_type=jnp.float32,
    )
    vrow = jnp.broadcast_to(pw, (NP, NP))
    jj = jax.lax.broadcasted_iota(jnp.int32, (NP, NP), 1)
    ii = jax.lax.broadcasted_iota(jnp.int32, (NP, NP), 0)
    beats = (vrow > vcol) | ((vrow == vcol) & (jj < ii))
    rank = jnp.sum(beats.astype(jnp.int32), axis=1, keepdims=True)

    rr = jax.lax.broadcasted_iota(jnp.int32, (NP, K), 1)
    sel_mask = rank == rr
    iidx = jax.lax.broadcasted_iota(jnp.int32, (NP, K), 0)
    idx_ref[0, :] = jnp.sum(jnp.where(sel_mask, iidx, 0), axis=0)[None, :]

    mask16 = sel_mask.astype(jnp.bfloat16)
    patches = tokens_ref[0, 1:N, :]
    hi = patches.astype(jnp.bfloat16)
    rest = (patches - hi.astype(jnp.float32)).astype(jnp.bfloat16)
    sel = jax.lax.dot_general(
        mask16, hi, _T_DN, preferred_element_type=jnp.float32,
    ) + jax.lax.dot_general(
        mask16, rest, _T_DN, preferred_element_type=jnp.float32,
    )
    merged_ref[0, 0] = tokens_ref[0, 0]
    merged_ref[0, 1:K + 1, :] = sel


@jax.jit
def kernel(tokens, attention_maps):
    merged, idx = pl.pallas_call(
        _merge_kernel,
        grid=(B,),
        in_specs=[
            pl.BlockSpec((1, H, N, N), lambda b: (b, 0, 0, 0)),
            pl.BlockSpec((1, N, D), lambda b: (b, 0, 0)),
        ],
        out_specs=[
            pl.BlockSpec((1, K + 1, D), lambda b: (b, 0, 0)),
            pl.BlockSpec((1, 1, K), lambda b: (b, 0, 0)),
        ],
        out_shape=[
            jax.ShapeDtypeStruct((B, K + 1, D), jnp.float32),
            jax.ShapeDtypeStruct((B, 1, K), jnp.int32),
        ],
        scratch_shapes=[
            pltpu.VMEM((N, N), jnp.float32),
        ],
        compiler_params=pltpu.CompilerParams(
            dimension_semantics=("arbitrary",),
        ),
    )(attention_maps, tokens)
    return merged, idx.reshape(B, K)
